# Initial kernel scaffold; baseline (speedup 1.0000x reference)
import jax
import jax.numpy as jnp
from jax.experimental import pallas as pl


def kernel(g_win, Wq, bq, Wk, bk):
    raise NotImplementedError("write your pallas kernel here")


# speedup vs baseline: 24.3772x; 24.3772x over previous
import functools

import jax
import jax.numpy as jnp
from jax.experimental import pallas as pl

QK = 96
NWIN = 1024
K = 4
SCALE = QK ** (-0.5)


def _body(g_ref, wq_ref, bq_ref, wk_ref, bk_ref, w_ref, i_ref):
    g = g_ref[0]
    q = jax.lax.dot_general(g, wq_ref[...], (((1,), (1,)), ((), ())),
                            preferred_element_type=jnp.float32) + bq_ref[0]
    k = jax.lax.dot_general(g, wk_ref[...], (((1,), (1,)), ((), ())),
                            preferred_element_type=jnp.float32) + bk_ref[0]
    attn = jax.lax.dot_general(q * SCALE, k, (((1,), (1,)), ((), ())),
                               preferred_element_type=jnp.float32)
    col = jax.lax.broadcasted_iota(jnp.int32, (NWIN, NWIN), 1)
    x = attn
    vals, idxs = [], []
    for _ in range(K):
        m = jnp.max(x, axis=-1, keepdims=True)
        am = jnp.min(jnp.where(x == m, col, NWIN), axis=-1, keepdims=True)
        vals.append(m)
        idxs.append(am)
        x = jnp.where(col == am, -jnp.inf, x)
    v = jnp.concatenate(vals, axis=-1)
    e = jnp.exp(v - vals[0])
    w = e / jnp.sum(e, axis=-1, keepdims=True)
    w_ref[0] = w
    i_ref[0] = jnp.concatenate(idxs, axis=-1)


@jax.jit
def kernel(g_win, Wq, bq, Wk, bk):
    B = g_win.shape[0]
    grid_spec = pl.GridSpec(
        grid=(B,),
        in_specs=[
            pl.BlockSpec((1, NWIN, QK), lambda b: (b, 0, 0)),
            pl.BlockSpec((QK, QK), lambda b: (0, 0)),
            pl.BlockSpec((1, QK), lambda b: (0, 0)),
            pl.BlockSpec((QK, QK), lambda b: (0, 0)),
            pl.BlockSpec((1, QK), lambda b: (0, 0)),
        ],
        out_specs=[
            pl.BlockSpec((1, NWIN, K), lambda b: (b, 0, 0)),
            pl.BlockSpec((1, NWIN, K), lambda b: (b, 0, 0)),
        ],
    )
    r_weight, topk_index = pl.pallas_call(
        _body,
        grid_spec=grid_spec,
        out_shape=[
            jax.ShapeDtypeStruct((B, NWIN, K), jnp.float32),
            jax.ShapeDtypeStruct((B, NWIN, K), jnp.int32),
        ],
    )(g_win, Wq, bq.reshape(1, QK), Wk, bk.reshape(1, QK))
    return (r_weight, topk_index)


# speedup vs baseline: 29.4714x; 1.2090x over previous
import functools

import jax
import jax.numpy as jnp
from jax.experimental import pallas as pl

QK = 96
NWIN = 1024
K = 4
SCALE = QK ** (-0.5)


def _body(g_ref, wq_ref, bq_ref, wk_ref, bk_ref, w_ref, i_ref):
    g = g_ref[0]
    q = jax.lax.dot_general(g, wq_ref[...], (((1,), (1,)), ((), ())),
                            preferred_element_type=jnp.float32) + bq_ref[0]
    k = jax.lax.dot_general(g, wk_ref[...], (((1,), (1,)), ((), ())),
                            preferred_element_type=jnp.float32) + bk_ref[0]
    attn = jax.lax.dot_general(q * SCALE, k, (((1,), (1,)), ((), ())),
                               preferred_element_type=jnp.float32)
    col = jax.lax.broadcasted_iota(
        jnp.int32, (NWIN, NWIN), 1).astype(jnp.float32)
    x = attn
    vals, idxs = [], []
    for _ in range(K):
        m = jnp.max(x, axis=-1, keepdims=True)
        am = jnp.min(jnp.where(x == m, col, float(NWIN)),
                     axis=-1, keepdims=True)
        vals.append(m)
        idxs.append(am)
        x = jnp.where(col == am, -jnp.inf, x)
    v = jnp.concatenate(vals, axis=-1)
    e = jnp.exp(v - vals[0])
    w = e / jnp.sum(e, axis=-1, keepdims=True)
    w_ref[0] = w
    i_ref[0] = jnp.concatenate(idxs, axis=-1).astype(jnp.int32)


@jax.jit
def kernel(g_win, Wq, bq, Wk, bk):
    B = g_win.shape[0]
    grid_spec = pl.GridSpec(
        grid=(B,),
        in_specs=[
            pl.BlockSpec((1, NWIN, QK), lambda b: (b, 0, 0)),
            pl.BlockSpec((QK, QK), lambda b: (0, 0)),
            pl.BlockSpec((1, QK), lambda b: (0, 0)),
            pl.BlockSpec((QK, QK), lambda b: (0, 0)),
            pl.BlockSpec((1, QK), lambda b: (0, 0)),
        ],
        out_specs=[
            pl.BlockSpec((1, NWIN, K), lambda b: (b, 0, 0)),
            pl.BlockSpec((1, NWIN, K), lambda b: (b, 0, 0)),
        ],
    )
    r_weight, topk_index = pl.pallas_call(
        _body,
        grid_spec=grid_spec,
        out_shape=[
            jax.ShapeDtypeStruct((B, NWIN, K), jnp.float32),
            jax.ShapeDtypeStruct((B, NWIN, K), jnp.int32),
        ],
    )(g_win, Wq, bq.reshape(1, QK), Wk, bk.reshape(1, QK))
    return (r_weight, topk_index)
